# baseline (speedup 1.0000x reference)
import functools

import jax
import jax.numpy as jnp
from jax import lax
from jax.experimental import pallas as pl
from jax.experimental.pallas import tpu as pltpu
from jax.experimental.pallas import tpu_sc as plsc

_MU = 3.5
_BATCH = 16384
_K = 90
_IDX_CHUNK = 128


def _dot_chunks():
    return (0, 16, 32, 48, 64, 74)


@functools.lru_cache(maxsize=None)
def _build(n_users, n_items, k, batch, idx_dtype):
    try:
        info = plsc.get_sparse_core_info()
        nc, ns = info.num_cores, info.num_subcores
    except Exception:
        nc, ns = 2, 16
    nw = nc * ns
    bpw = batch // nw
    n_chunks = bpw // _IDX_CHUNK
    assert bpw * nw == batch and n_chunks * _IDX_CHUNK == bpw

    mesh = plsc.VectorSubcoreMesh(core_axis_name="c", subcore_axis_name="s",
                                  num_cores=nc, num_subcores=ns)

    @functools.partial(
        pl.kernel,
        mesh=mesh,
        compiler_params=pltpu.CompilerParams(needs_layout_passes=False,
                                             use_tc_tiling_on_sc=False),
        out_type=jax.ShapeDtypeStruct((batch,), jnp.float32),
        scratch_types=[
            [pltpu.VMEM((_IDX_CHUNK,), jnp.int32) for _ in range(n_chunks)],
            [pltpu.VMEM((_IDX_CHUNK,), jnp.int32) for _ in range(n_chunks)],
            pltpu.VMEM((bpw, k), jnp.float32),
            pltpu.VMEM((bpw, k), jnp.float32),
            pltpu.VMEM((bpw,), jnp.float32),
            pltpu.VMEM((bpw,), jnp.float32),
            pltpu.VMEM((bpw,), jnp.float32),
            pltpu.SemaphoreType.DMA,
        ],
    )
    def lfm(uidx_hbm, iidx_hbm, p_hbm, q_hbm, bu_hbm, bi_hbm, out_hbm,
            uidx_v, iidx_v, p_rows, q_rows, bu_v, bi_v, out_v, sem):
        wid = lax.axis_index("s") * nc + lax.axis_index("c")
        base = pl.multiple_of(wid * bpw, _IDX_CHUNK)

        for c in range(n_chunks):
            pltpu.sync_copy(uidx_hbm.at[pl.ds(base + c * _IDX_CHUNK, _IDX_CHUNK)],
                            uidx_v[c])
            pltpu.sync_copy(iidx_hbm.at[pl.ds(base + c * _IDX_CHUNK, _IDX_CHUNK)],
                            iidx_v[c])

        copies = []
        for c in range(n_chunks):
            dst = pl.ds(c * _IDX_CHUNK, _IDX_CHUNK)
            copies.append(pltpu.async_copy(p_hbm.at[uidx_v[c]],
                                           p_rows.at[dst], sem))
            copies.append(pltpu.async_copy(q_hbm.at[iidx_v[c]],
                                           q_rows.at[dst], sem))
            copies.append(pltpu.async_copy(bu_hbm.at[uidx_v[c]],
                                           bu_v.at[dst], sem))
            copies.append(pltpu.async_copy(bi_hbm.at[iidx_v[c]],
                                           bi_v.at[dst], sem))
        for cp in copies:
            cp.wait()

        lane = lax.iota(jnp.int32, 16)

        def group_body(g, carry):
            ids = g * 16 + lane
            acc = jnp.zeros((16,), jnp.float32)
            kvec = jnp.zeros((16,), jnp.int32)
            for _ in range(k):
                pk = plsc.load_gather(p_rows, [ids, kvec])
                qk = plsc.load_gather(q_rows, [ids, kvec])
                acc = acc + pk * qk
                kvec = kvec + 1
            sl = pl.ds(g * 16, 16)
            out_v[sl] = acc + bu_v[sl] + bi_v[sl] + _MU
            return carry

        lax.fori_loop(0, bpw // 16, group_body, 0)

        pltpu.sync_copy(out_v, out_hbm.at[pl.ds(base, bpw)])

    return lfm


def kernel(user_idx, item_idx, P, Q, b_u, b_i):
    fn = _build(P.shape[0], Q.shape[0], P.shape[1], user_idx.shape[0],
                jnp.int32)
    return fn(user_idx.astype(jnp.int32), item_idx.astype(jnp.int32),
              P, Q, b_u.reshape(-1), b_i.reshape(-1))
